# Initial kernel scaffold; baseline (speedup 1.0000x reference)
import jax
import jax.numpy as jnp
from jax.experimental import pallas as pl


def kernel(hidden_states, probs, routing_map):
    raise NotImplementedError("write your pallas kernel here")


# speedup vs baseline: 4.0076x; 4.0076x over previous
import jax
import jax.numpy as jnp
from jax.experimental import pallas as pl

_E = 8
_BLK = 512


def _body(h_ref, p_ref, m_ref, out_ref, tpe_ref):
    i = pl.program_id(0)
    m = m_ref[...]
    scale = jnp.sum(p_ref[...] * m, axis=1, keepdims=True)
    out_ref[...] = h_ref[...] * scale
    part = jnp.sum(m, axis=0, keepdims=True)

    @pl.when(i == 0)
    def _init():
        tpe_ref[...] = part

    @pl.when(i != 0)
    def _acc():
        tpe_ref[...] += part


def kernel(hidden_states, probs, routing_map):
    hidden_shape = hidden_states.shape
    H = hidden_shape[-1]
    T = hidden_states.size // H
    h = hidden_states.reshape(T, H)
    mask_f = routing_map.astype(jnp.float32)

    out, tpe = pl.pallas_call(
        _body,
        grid=(T // _BLK,),
        in_specs=[
            pl.BlockSpec((_BLK, H), lambda i: (i, 0)),
            pl.BlockSpec((_BLK, _E), lambda i: (i, 0)),
            pl.BlockSpec((_BLK, _E), lambda i: (i, 0)),
        ],
        out_specs=[
            pl.BlockSpec((_BLK, H), lambda i: (i, 0)),
            pl.BlockSpec((1, _E), lambda i: (0, 0)),
        ],
        out_shape=[
            jax.ShapeDtypeStruct((T, H), jnp.float32),
            jax.ShapeDtypeStruct((1, _E), jnp.float32),
        ],
    )(h, probs, mask_f)

    tokens_per_expert = tpe.reshape(_E).astype(jnp.int64)
    return out.reshape(hidden_shape), tokens_per_expert


# speedup vs baseline: 4.0286x; 1.0053x over previous
import jax
import jax.numpy as jnp
from jax.experimental import pallas as pl

_E = 8
_BLK = 1024


def _body(h_ref, p_ref, m_ref, out_ref, tpe_ref):
    i = pl.program_id(0)
    m = m_ref[...]
    scale = jnp.sum(p_ref[...] * m, axis=1, keepdims=True)
    out_ref[...] = h_ref[...] * scale
    part = jnp.sum(m, axis=0, keepdims=True)

    @pl.when(i == 0)
    def _init():
        tpe_ref[...] = part

    @pl.when(i != 0)
    def _acc():
        tpe_ref[...] += part


def kernel(hidden_states, probs, routing_map):
    hidden_shape = hidden_states.shape
    H = hidden_shape[-1]
    T = hidden_states.size // H
    h = hidden_states.reshape(T, H)
    mask_f = routing_map.astype(jnp.float32)

    out, tpe = pl.pallas_call(
        _body,
        grid=(T // _BLK,),
        in_specs=[
            pl.BlockSpec((_BLK, H), lambda i: (i, 0)),
            pl.BlockSpec((_BLK, _E), lambda i: (i, 0)),
            pl.BlockSpec((_BLK, _E), lambda i: (i, 0)),
        ],
        out_specs=[
            pl.BlockSpec((_BLK, H), lambda i: (i, 0)),
            pl.BlockSpec((1, _E), lambda i: (0, 0)),
        ],
        out_shape=[
            jax.ShapeDtypeStruct((T, H), jnp.float32),
            jax.ShapeDtypeStruct((1, _E), jnp.float32),
        ],
    )(h, probs, mask_f)

    tokens_per_expert = tpe.reshape(_E).astype(jnp.int64)
    return out.reshape(hidden_shape), tokens_per_expert
